# baseline (speedup 1.0000x reference)
import jax
import jax.numpy as jnp
from jax.experimental import pallas as pl

_NL = 5
_N = 10000


def _mlp_kernel(g_ref, w1_ref, b1_ref, w2_ref, b2_ref, o_ref):
    x = jnp.maximum(g_ref[...] @ w1_ref[...] + b1_ref[...][None, :], 0.0)
    o_ref[...] = x @ w2_ref[...] + b2_ref[...][None, :]


def kernel(node_feat, edge_index, W_embed, b_embed, W_fc, attn_l, attn_r, bias_g, W1, b1, W2, b2):
    src = edge_index[0]
    dst = edge_index[1]
    h = node_feat @ W_embed + b_embed
    for i in range(_NL):
        ft = h @ W_fc[i]
        el = jnp.sum(ft * attn_l[i], axis=-1)
        er = jnp.sum(ft * attn_r[i], axis=-1)
        e = el[src] + er[dst]
        e = jnp.where(e > 0, e, 0.2 * e)
        m = jax.ops.segment_max(e, dst, num_segments=_N)
        ex = jnp.exp(e - m[dst])
        s = jax.ops.segment_sum(ex, dst, num_segments=_N)
        a = ex / s[dst]
        out = jax.ops.segment_sum(ft[src] * a[:, None], dst, num_segments=_N)
        out = out + h + bias_g[i]
        if i < _NL - 1:
            out = jnp.where(out > 0, out, jnp.expm1(out))
        h = out
    g = jnp.sum(h, axis=0, keepdims=True)
    return pl.pallas_call(
        _mlp_kernel,
        out_shape=jax.ShapeDtypeStruct((1, 1), jnp.float32),
    )(g, W1, b1, W2, b2)


# speedup vs baseline: 12.5471x; 12.5471x over previous
import jax
import jax.numpy as jnp
from jax import lax
from jax.experimental import pallas as pl
from jax.experimental.pallas import tpu as pltpu
from jax.experimental.pallas import tpu_sc as plsc

_N = 10000
_NP = 10240
_E = 320000
_H = 200
_HF = 104
_HG = 128
_NL = 5
_NC = 2
_NS = 16
_NW = _NC * _NS
_EPT1 = _E // _NW
_NV1 = _EPT1 // 16
_EPT3 = _E // _NS
_C = 80
_NKC = 10
_BLKE = _C * _NKC
_NBLK = _EPT3 // _BLKE
_BLK = 256
_GRID = _NP // _BLK
_SLICE = _NP // _NS
_NEG = -3.0e38

_f32 = jnp.float32
_i32 = jnp.int32

_mesh = plsc.VectorSubcoreMesh(core_axis_name="c", subcore_axis_name="s")
_sc_params = pltpu.CompilerParams(
    needs_layout_passes=False, use_tc_tiling_on_sc=False)


def _vgather(x, i):
    dn = lax.GatherDimensionNumbers(
        offset_dims=(), collapsed_slice_dims=(0,), start_index_map=(0,))
    return lax.gather(x, i[:, None], dn, slice_sizes=(1,),
                      mode=lax.GatherScatterMode.PROMISE_IN_BOUNDS)


def _s1_body(el_hbm, er_hbm, src_hbm, dst_hbm, e_out, m_part,
             el_t, er_t, m_l, src_t, dst_t, e_t):
    cid = lax.axis_index("c")
    sid = lax.axis_index("s")
    wid = sid * _NC + cid
    pltpu.sync_copy(el_hbm, el_t)
    pltpu.sync_copy(er_hbm, er_t)
    pltpu.sync_copy(src_hbm.at[wid], src_t)
    pltpu.sync_copy(dst_hbm.at[wid], dst_t)

    neg = jnp.full((16,), _NEG, _f32)

    def init_b(i, carry):
        m_l[pl.ds(i * 16, 16)] = neg
        return carry

    lax.fori_loop(0, _NP // 16, init_b, 0)

    idx = lax.iota(_i32, 16)

    def edge_b(i, carry):
        sv = src_t[i, :]
        dv = dst_t[i, :]
        ev = plsc.load_gather(el_t, [sv]) + plsc.load_gather(er_t, [dv])
        ev = jnp.where(ev > 0, ev, 0.2 * ev)
        e_t[i, :] = ev
        ks, es = plsc.sort_key_val(dv, ev)
        run = es
        for d in (1, 2, 4, 8):
            sl = jnp.maximum(idx - d, 0)
            ks_sh = _vgather(ks, sl)
            run_sh = _vgather(run, sl)
            ok = (ks_sh == ks) & (idx >= d)
            run = jnp.where(ok, jnp.maximum(run, run_sh), run)
        ks_n = _vgather(ks, jnp.minimum(idx + 1, 15))
        last = (ks_n != ks) | (idx == 15)
        old = plsc.load_gather(m_l, [ks])
        plsc.store_scatter(m_l, [ks], jnp.maximum(old, run), mask=last)
        return carry

    lax.fori_loop(0, _NV1, edge_b, 0)
    pltpu.sync_copy(e_t, e_out.at[wid])
    pltpu.sync_copy(m_l, m_part.at[wid])


_s1_call = pl.kernel(
    _s1_body,
    out_type=(
        jax.ShapeDtypeStruct((_NW, _NV1, 16), _f32),
        jax.ShapeDtypeStruct((_NW, _NP), _f32),
    ),
    mesh=_mesh,
    compiler_params=_sc_params,
    scratch_types=[
        pltpu.VMEM((_NP,), _f32),
        pltpu.VMEM((_NP,), _f32),
        pltpu.VMEM((_NP,), _f32),
        pltpu.VMEM((_NV1, 16), _i32),
        pltpu.VMEM((_NV1, 16), _i32),
        pltpu.VMEM((_NV1, 16), _f32),
    ],
)


def _s3_body(ft0_hbm, ft1_hbm, e_hbm, src_hbm, dst_hbm, mp_hbm,
             out_part, s_part,
             m_t, sb, db, eb, ex_b, rows_g, rows_s, mbuf,
             out_sh, s_sh, m_sh,
             bl0, bl1, gs0, gs1, ws0, ws1, qs0, qs1):
    cid = lax.axis_index("c")
    sid = lax.axis_index("s")
    base = sid * _SLICE
    col0 = 8 * cid

    zv = jnp.zeros((16,), _f32)

    def zrow(r, carry):
        for j in range(6):
            rows_s[0, r, pl.ds(j * 16, 16)] = zv
        rows_s[0, r, pl.ds(_HF - 16, 16)] = zv
        return carry

    lax.fori_loop(0, _C, zrow, 0)

    def zb(j, carry):
        mbuf[0, pl.ds(j * 16, 16)] = zv
        return carry

    lax.fori_loop(0, 10, zb, 0)

    for k in range(_SLICE // _C):
        pltpu.sync_copy(rows_s.at[0], out_sh.at[pl.ds(base + k * _C, _C), :])

    @pl.when(cid == 0)
    def _():
        for q in range(4):
            pltpu.sync_copy(mbuf.at[0], s_sh.at[pl.ds(base + q * 160, 160)])

    for q in range(4):
        off = base + q * 160
        pltpu.sync_copy(mp_hbm.at[:, pl.ds(off, 160)], mbuf)

        def mred(j, carry):
            acc = mbuf[0, pl.ds(j * 16, 16)]
            for r in range(1, _NW):
                acc = jnp.maximum(acc, mbuf[r, pl.ds(j * 16, 16)])
            mbuf[0, pl.ds(j * 16, 16)] = acc
            return carry

        lax.fori_loop(0, 10, mred, 0)
        pltpu.sync_copy(mbuf.at[0], m_sh.at[pl.ds(off, 160)])

    plsc.subcore_barrier()
    pltpu.sync_copy(m_sh, m_t)

    blsems = (bl0, bl1)
    gsems = (gs0, gs1)
    wsems = (ws0, ws1)
    qsems = (qs0, qs1)

    def issue_block_load(b, bslot):
        pltpu.async_copy(src_hbm.at[sid, b], sb.at[bslot], blsems[bslot])
        pltpu.async_copy(dst_hbm.at[sid, b], db.at[bslot], blsems[bslot])
        pltpu.async_copy(e_hbm.at[sid, b], eb.at[bslot], blsems[bslot])

    def wait_block_load(b, bslot):
        pltpu.make_async_copy(src_hbm.at[sid, b], sb.at[bslot],
                              blsems[bslot]).wait()
        pltpu.make_async_copy(dst_hbm.at[sid, b], db.at[bslot],
                              blsems[bslot]).wait()
        pltpu.make_async_copy(e_hbm.at[sid, b], eb.at[bslot],
                              blsems[bslot]).wait()

    def issue_gather(bslot, k, kslot):
        @pl.when(cid == 0)
        def _():
            pltpu.async_copy(ft0_hbm.at[sb.at[bslot, k]], rows_g.at[kslot],
                             gsems[kslot])

        @pl.when(cid == 1)
        def _():
            pltpu.async_copy(ft1_hbm.at[sb.at[bslot, k]], rows_g.at[kslot],
                             gsems[kslot])

    def wait_gather(bslot, k, kslot):
        pltpu.make_async_copy(ft0_hbm.at[sb.at[bslot, k]], rows_g.at[kslot],
                              gsems[kslot]).wait()

    def process_chunk(b, bslot, k, kslot):
        wait_gather(bslot, k, kslot)

        @pl.when(k < _NKC - 1)
        def _():
            issue_gather(bslot, k + 1, 1 - kslot)

        @pl.when((b > 0) | (k >= 2))
        def _():
            pltpu.make_async_copy(
                rows_s.at[kslot], out_sh.at[db.at[bslot, k]],
                wsems[kslot]).wait()

            @pl.when(cid == 0)
            def _():
                pltpu.make_async_copy(
                    ex_b.at[kslot], s_sh.at[db.at[bslot, k]],
                    qsems[kslot]).wait()

        def exg(g, carry):
            dv = db[bslot, k, pl.ds(g * 16, 16)]
            ev = eb[bslot, k, pl.ds(g * 16, 16)]
            mv = plsc.load_gather(m_t, [dv])
            ex_b[kslot, pl.ds(g * 16, 16)] = jnp.exp(ev - mv)
            return carry

        lax.fori_loop(0, _C // 16, exg, 0)

        def grp(g, carry):
            exv = ex_b[kslot, pl.ds(g * 16, 16)]
            for l in range(16):
                row = g * 16 + l
                a_l = _vgather(exv, jnp.full((16,), l, _i32))
                for j in range(7):
                    jj = (_HF - 16) if j == 6 else j * 16
                    rows_s[kslot, row, pl.ds(jj, 16)] = (
                        a_l * rows_g[kslot, row, pl.ds(col0 + jj, 16)])
            return carry

        lax.fori_loop(0, _C // 16, grp, 0)

        pltpu.async_copy(rows_s.at[kslot], out_sh.at[db.at[bslot, k]],
                         wsems[kslot], add=True)

        @pl.when(cid == 0)
        def _():
            pltpu.async_copy(ex_b.at[kslot], s_sh.at[db.at[bslot, k]],
                             qsems[kslot], add=True)

    def run_block(b, bslot):
        wait_block_load(b, bslot)

        @pl.when(b + 1 < _NBLK)
        def _():
            issue_block_load(b + 1, 1 - bslot)

        issue_gather(bslot, 0, 0)

        def kbody(k, carry):
            @pl.when(k % 2 == 0)
            def _():
                process_chunk(b, bslot, k, 0)

            @pl.when(k % 2 == 1)
            def _():
                process_chunk(b, bslot, k, 1)

            return carry

        lax.fori_loop(0, _NKC, kbody, 0)

    issue_block_load(0, 0)

    def bbody(b, carry):
        @pl.when(b % 2 == 0)
        def _():
            run_block(b, 0)

        @pl.when(b % 2 == 1)
        def _():
            run_block(b, 1)

        return carry

    lax.fori_loop(0, _NBLK, bbody, 0)

    for kslot in range(2):
        pltpu.make_async_copy(
            rows_s.at[kslot], out_sh.at[db.at[0, 0]], wsems[kslot]).wait()

        @pl.when(cid == 0)
        def _():
            pltpu.make_async_copy(
                ex_b.at[kslot], s_sh.at[db.at[0, 0]], qsems[kslot]).wait()

    plsc.subcore_barrier()

    pltpu.sync_copy(out_sh.at[pl.ds(base, _SLICE), :],
                    out_part.at[cid, pl.ds(base, _SLICE), :])

    @pl.when(cid == 0)
    def _():
        pltpu.sync_copy(s_sh.at[pl.ds(base, _SLICE)],
                        s_part.at[pl.ds(base, _SLICE)])


_s3_call = pl.kernel(
    _s3_body,
    out_type=(
        jax.ShapeDtypeStruct((_NC, _NP, _HF), _f32),
        jax.ShapeDtypeStruct((_NP,), _f32),
    ),
    mesh=_mesh,
    compiler_params=_sc_params,
    scratch_types=[
        pltpu.VMEM((_NP,), _f32),
        pltpu.VMEM((2, _NKC, _C), _i32),
        pltpu.VMEM((2, _NKC, _C), _i32),
        pltpu.VMEM((2, _NKC, _C), _f32),
        pltpu.VMEM((2, _C), _f32),
        pltpu.VMEM((2, _C, _HG), _f32),
        pltpu.VMEM((2, _C, _HF), _f32),
        pltpu.VMEM((_NW, 160), _f32),
        pltpu.VMEM_SHARED((_NP, _HF), _f32),
        pltpu.VMEM_SHARED((_NP,), _f32),
        pltpu.VMEM_SHARED((_NP,), _f32),
        pltpu.SemaphoreType.DMA,
        pltpu.SemaphoreType.DMA,
        pltpu.SemaphoreType.DMA,
        pltpu.SemaphoreType.DMA,
        pltpu.SemaphoreType.DMA,
        pltpu.SemaphoreType.DMA,
        pltpu.SemaphoreType.DMA,
        pltpu.SemaphoreType.DMA,
    ],
)


def _row_mask(i, x):
    rows = i * _BLK + lax.broadcasted_iota(_i32, (_BLK, 1), 0)
    return jnp.where(rows < _N, x, 0.0)


def _head_tail(h, w_ref, al_ref, ar_ref, h_ref, f0_ref, f1_ref,
               el_ref, er_ref):
    h_ref[...] = h
    ft = h @ w_ref[...]
    f0_ref[...] = ft[:, :_HG]
    f1_ref[...] = ft[:, 96:96 + _HG]
    el_ref[...] = jnp.sum(ft * al_ref[...][None, :], axis=1)
    er_ref[...] = jnp.sum(ft * ar_ref[...][None, :], axis=1)


def _embed_body(nf_ref, we_ref, be_ref, w_ref, al_ref, ar_ref,
                h_ref, f0_ref, f1_ref, el_ref, er_ref):
    i = pl.program_id(0)
    h = nf_ref[...] @ we_ref[...] + be_ref[...][None, :]
    h = _row_mask(i, h)
    _head_tail(h, w_ref, al_ref, ar_ref, h_ref, f0_ref, f1_ref,
               el_ref, er_ref)


def _finalize(op_ref, sp_ref, hp_ref, b_ref, elu):
    s = sp_ref[...]
    rin = jnp.where(s > 0, 1.0 / s, 0.0)
    op = jnp.concatenate((op_ref[0], op_ref[1][:, :_H - _HF]), axis=1)
    o = op * rin[:, None] + hp_ref[...] + b_ref[...][None, :]
    if elu:
        o = jnp.where(o > 0, o, jnp.exp(o) - 1.0)
    return o


def _layer_body(op_ref, sp_ref, hp_ref, b_ref, w_ref, al_ref, ar_ref,
                h_ref, f0_ref, f1_ref, el_ref, er_ref):
    i = pl.program_id(0)
    h = _row_mask(i, _finalize(op_ref, sp_ref, hp_ref, b_ref, True))
    _head_tail(h, w_ref, al_ref, ar_ref, h_ref, f0_ref, f1_ref,
               el_ref, er_ref)


def _final_body(op_ref, sp_ref, hp_ref, b_ref, w1_ref, b1_ref, w2_ref, b2_ref,
                y_ref, g_acc):
    i = pl.program_id(0)
    h = _row_mask(i, _finalize(op_ref, sp_ref, hp_ref, b_ref, False))
    g = jnp.sum(h, axis=0, keepdims=True)

    @pl.when(i == 0)
    def _():
        g_acc[...] = g

    @pl.when(i > 0)
    def _():
        g_acc[...] += g

    @pl.when(i == _GRID - 1)
    def _():
        x = jnp.maximum(g_acc[...] @ w1_ref[...] + b1_ref[...][None, :], 0.0)
        y_ref[...] = x @ w2_ref[...] + b2_ref[...][None, :]


def _head_out_specs():
    return (
        [
            jax.ShapeDtypeStruct((_NP, _H), _f32),
            jax.ShapeDtypeStruct((_NP, _HG), _f32),
            jax.ShapeDtypeStruct((_NP, _HG), _f32),
            jax.ShapeDtypeStruct((_NP,), _f32),
            jax.ShapeDtypeStruct((_NP,), _f32),
        ],
        [
            pl.BlockSpec((_BLK, _H), lambda i: (i, 0)),
            pl.BlockSpec((_BLK, _HG), lambda i: (i, 0)),
            pl.BlockSpec((_BLK, _HG), lambda i: (i, 0)),
            pl.BlockSpec((_BLK,), lambda i: (i,)),
            pl.BlockSpec((_BLK,), lambda i: (i,)),
        ],
    )


def _embed_call(nf, we, be, w, al, ar):
    out_shape, out_specs = _head_out_specs()
    return pl.pallas_call(
        _embed_body,
        grid=(_GRID,),
        in_specs=[
            pl.BlockSpec((_BLK, 128), lambda i: (i, 0)),
            pl.BlockSpec((128, _H), lambda i: (0, 0)),
            pl.BlockSpec((_H,), lambda i: (0,)),
            pl.BlockSpec((_H, 256), lambda i: (0, 0)),
            pl.BlockSpec((256,), lambda i: (0,)),
            pl.BlockSpec((256,), lambda i: (0,)),
        ],
        out_specs=out_specs,
        out_shape=out_shape,
    )(nf, we, be, w, al, ar)


def _layer_call(op, sp, hp, b, w, al, ar):
    out_shape, out_specs = _head_out_specs()
    return pl.pallas_call(
        _layer_body,
        grid=(_GRID,),
        in_specs=[
            pl.BlockSpec((_NC, _BLK, _HF), lambda i: (0, i, 0)),
            pl.BlockSpec((_BLK,), lambda i: (i,)),
            pl.BlockSpec((_BLK, _H), lambda i: (i, 0)),
            pl.BlockSpec((_H,), lambda i: (0,)),
            pl.BlockSpec((_H, 256), lambda i: (0, 0)),
            pl.BlockSpec((256,), lambda i: (0,)),
            pl.BlockSpec((256,), lambda i: (0,)),
        ],
        out_specs=out_specs,
        out_shape=out_shape,
    )(op, sp, hp, b, w, al, ar)


def _final_call(op, sp, hp, b, w1, b1, w2, b2):
    return pl.pallas_call(
        _final_body,
        grid=(_GRID,),
        in_specs=[
            pl.BlockSpec((_NC, _BLK, _HF), lambda i: (0, i, 0)),
            pl.BlockSpec((_BLK,), lambda i: (i,)),
            pl.BlockSpec((_BLK, _H), lambda i: (i, 0)),
            pl.BlockSpec((_H,), lambda i: (0,)),
            pl.BlockSpec((_H, 1024), lambda i: (0, 0)),
            pl.BlockSpec((1024,), lambda i: (0,)),
            pl.BlockSpec((1024, 1), lambda i: (0, 0)),
            pl.BlockSpec((1,), lambda i: (0,)),
        ],
        out_specs=pl.BlockSpec((1, 1), lambda i: (0, 0)),
        out_shape=jax.ShapeDtypeStruct((1, 1), _f32),
        scratch_shapes=[pltpu.VMEM((1, _H), _f32)],
    )(op, sp, hp, b, w1, b1, w2, b2)


def kernel(node_feat, edge_index, W_embed, b_embed, W_fc, attn_l, attn_r,
           bias_g, W1, b1, W2, b2):
    nf = jnp.zeros((_NP, node_feat.shape[1]), _f32).at[:_N].set(node_feat)
    src = edge_index[0]
    dst = edge_index[1]
    src1 = src.reshape(_NW, _NV1, 16)
    dst1 = dst.reshape(_NW, _NV1, 16)
    src4 = src.reshape(_NS, _NBLK, _NKC, _C)
    dst4 = dst.reshape(_NS, _NBLK, _NKC, _C)
    wfc = jnp.zeros((_NL, _H, 256), _f32).at[:, :, :_H].set(W_fc)
    al = jnp.zeros((_NL, 256), _f32).at[:, :_H].set(attn_l)
    ar = jnp.zeros((_NL, 256), _f32).at[:, :_H].set(attn_r)

    h, f0, f1, el, er = _embed_call(nf, W_embed, b_embed, wfc[0], al[0], ar[0])
    y = None
    for i in range(_NL):
        e1, m_part = _s1_call(el, er, src1, dst1)
        e4 = e1.reshape(_NS, _NBLK, _NKC, _C)
        op, sp = _s3_call(f0, f1, e4, src4, dst4, m_part)
        if i < _NL - 1:
            h, f0, f1, el, er = _layer_call(op, sp, h, bias_g[i],
                                            wfc[i + 1], al[i + 1], ar[i + 1])
        else:
            y = _final_call(op, sp, h, bias_g[i], W1, b1, W2, b2)
    return y
